# baseline (speedup 1.0000x reference)
import functools

import jax
import jax.numpy as jnp
from jax import lax
from jax.experimental import pallas as pl
from jax.experimental.pallas import tpu as pltpu
from jax.experimental.pallas import tpu_sc as plsc

B = 16384
D = 128
NC = 2
NS = 16
NW = NC * NS
BPW = B // NW
C = 128
NCHUNK = BPW // C
LANES = 16


def kernel(row, col, dir, row_emb, col_emb, dir_emb):
    row2 = row.astype(jnp.int32).reshape(B // C, C)
    col2 = col.astype(jnp.int32).reshape(B // C, C)
    dir2 = dir.astype(jnp.int32).reshape(B // C, C)

    mesh = plsc.VectorSubcoreMesh(core_axis_name="c", subcore_axis_name="s")

    @functools.partial(
        pl.kernel,
        mesh=mesh,
        out_type=jax.ShapeDtypeStruct((B, D), jnp.float32),
        scratch_types=[
            pltpu.VMEM((NCHUNK, C), jnp.int32),
            pltpu.VMEM((NCHUNK, C), jnp.int32),
            pltpu.VMEM((NCHUNK, C), jnp.int32),
            pltpu.VMEM((C, D), jnp.float32),
            pltpu.VMEM((C, D), jnp.float32),
            pltpu.VMEM((C, D), jnp.float32),
            pltpu.SemaphoreType.DMA,
        ],
    )
    def k(row_hbm, col_hbm, dir_hbm, re_hbm, ce_hbm, de_hbm, out_hbm,
          ri_v, ci_v, di_v, a_v, b_v, c_v, sem):
        wid = lax.axis_index("s") * NC + lax.axis_index("c")
        chunk0 = wid * NCHUNK
        pltpu.sync_copy(row_hbm.at[pl.ds(chunk0, NCHUNK)], ri_v)
        pltpu.sync_copy(col_hbm.at[pl.ds(chunk0, NCHUNK)], ci_v)
        pltpu.sync_copy(dir_hbm.at[pl.ds(chunk0, NCHUNK)], di_v)

        @pl.loop(0, NCHUNK)
        def _(ch):
            cp1 = pltpu.async_copy(re_hbm.at[ri_v.at[ch]], a_v, sem)
            cp2 = pltpu.async_copy(ce_hbm.at[ci_v.at[ch]], b_v, sem)
            cp3 = pltpu.async_copy(de_hbm.at[di_v.at[ch]], c_v, sem)
            cp1.wait()
            cp2.wait()
            cp3.wait()

            @pl.loop(0, C)
            def _(r):
                @pl.loop(0, D, step=LANES)
                def _(j):
                    s = pl.ds(j, LANES)
                    a_v[r, s] = a_v[r, s] + b_v[r, s] + c_v[r, s]

            off = (chunk0 + ch) * C
            pltpu.sync_copy(a_v, out_hbm.at[pl.ds(off, C)])

    return k(row2, col2, dir2, row_emb, col_emb, dir_emb)


# speedup vs baseline: 10.2679x; 10.2679x over previous
import functools

import jax
import jax.numpy as jnp
from jax import lax
from jax.experimental import pallas as pl
from jax.experimental.pallas import tpu as pltpu
from jax.experimental.pallas import tpu_sc as plsc

B = 16384
D = 128
L = 1000
NC = 2
NS = 16
NW = NC * NS
BPW = B // NW
C = 128
NCHUNK = BPW // C
LANES = 16


def _coldir_body(dir_ref, col_ref, o_ref):
    o_ref[...] = dir_ref[...][:, None, :] + col_ref[...][None, :, :]


def _build_coldir(col_emb, dir_emb):
    out = pl.pallas_call(
        _coldir_body,
        out_shape=jax.ShapeDtypeStruct((2, L, D), jnp.float32),
    )(dir_emb, col_emb)
    return out.reshape(2 * L, D)


def kernel(row, col, dir, row_emb, col_emb, dir_emb):
    coldir = _build_coldir(col_emb, dir_emb)
    row2 = row.astype(jnp.int32).reshape(B // C, C)
    col2 = col.astype(jnp.int32).reshape(B // C, C)
    dir2 = dir.astype(jnp.int32).reshape(B // C, C)

    mesh = plsc.VectorSubcoreMesh(core_axis_name="c", subcore_axis_name="s")

    @functools.partial(
        pl.kernel,
        mesh=mesh,
        out_type=jax.ShapeDtypeStruct((B, D), jnp.float32),
        scratch_types=[
            pltpu.VMEM((NCHUNK, C), jnp.int32),
            pltpu.VMEM((NCHUNK, C), jnp.int32),
            pltpu.VMEM((NCHUNK, C), jnp.int32),
            pltpu.VMEM((NCHUNK, C), jnp.int32),
            pltpu.VMEM((C, D), jnp.float32),
            pltpu.VMEM((C, D), jnp.float32),
            pltpu.VMEM((C, D), jnp.float32),
            pltpu.VMEM((C, D), jnp.float32),
            pltpu.SemaphoreType.DMA,
        ],
    )
    def k(row_hbm, col_hbm, dir_hbm, cd_hbm, re_hbm, out_hbm,
          ri_v, ci_v, di_v, cd_v, a0_v, a1_v, b0_v, b1_v, sem):
        wid = lax.axis_index("s") * NC + lax.axis_index("c")
        chunk0 = wid * NCHUNK
        pltpu.sync_copy(row_hbm.at[pl.ds(chunk0, NCHUNK)], ri_v)
        pltpu.sync_copy(col_hbm.at[pl.ds(chunk0, NCHUNK)], ci_v)
        pltpu.sync_copy(dir_hbm.at[pl.ds(chunk0, NCHUNK)], di_v)

        @pl.loop(0, NCHUNK)
        def _(r):
            for j in range(0, C, LANES):
                s = pl.ds(j, LANES)
                cd_v[r, s] = di_v[r, s] * L + ci_v[r, s]

        abufs = (a0_v, a1_v)
        bbufs = (b0_v, b1_v)

        def fire(ch):
            a = pltpu.async_copy(re_hbm.at[ri_v.at[ch]], abufs[ch % 2], sem)
            b = pltpu.async_copy(cd_hbm.at[cd_v.at[ch]], bbufs[ch % 2], sem)
            return a, b

        cps = fire(0)
        for ch in range(NCHUNK):
            nxt = fire(ch + 1) if ch + 1 < NCHUNK else None
            cps[0].wait()
            cps[1].wait()
            a_v, b_v = abufs[ch % 2], bbufs[ch % 2]

            @pl.loop(0, C)
            def _(r):
                for j in range(0, D, LANES):
                    s = pl.ds(j, LANES)
                    a_v[r, s] = a_v[r, s] + b_v[r, s]

            pltpu.sync_copy(a_v, out_hbm.at[pl.ds((chunk0 + ch) * C, C)])
            cps = nxt

    return k(row2, col2, dir2, coldir, row_emb)
